# baseline (speedup 1.0000x reference)
import functools

import jax
import jax.numpy as jnp
from jax.experimental import pallas as pl
from jax.experimental.pallas import tpu as pltpu

_NEG = -1e30


def _cgmn_body(x_ref, b_ref, prior_ref, em_ref, cmT_ref, ghWT_ref,
               ghb_ref, goW_ref, gob_ref, outWT_ref, outb_ref, out_ref,
               T_s, m_s, d_s, num_s, *, n_tiles, n_graphs):
    i = pl.program_id(0)
    G, M = T_s.shape
    F = num_s.shape[1]
    TN = x_ref.shape[2]

    @pl.when(i == 0)
    def _init():
        pr = prior_ref[...]
        pe = jnp.exp(pr - jnp.max(pr, axis=1, keepdims=True))
        p = pe / jnp.sum(pe, axis=1, keepdims=True)
        em = em_ref[...]
        ee = jnp.exp(em - jnp.max(em, axis=2, keepdims=True))
        ep = ee / jnp.sum(ee, axis=2, keepdims=True)
        T_s[...] = jnp.log(jnp.sum(p[:, :, None] * ep, axis=1))
        m_s[...] = jnp.full((n_graphs, 1), _NEG, jnp.float32)
        d_s[...] = jnp.zeros((n_graphs, 1), jnp.float32)
        num_s[...] = jnp.zeros((n_graphs, F), jnp.float32)

    xi = x_ref[0]
    bi = b_ref[0]

    miota = jax.lax.broadcasted_iota(jnp.int32, (M, TN), 0)
    ohM = (miota == xi).astype(jnp.float32)
    llT = jnp.dot(T_s[...], ohM, preferred_element_type=jnp.float32)

    P = cmT_ref.shape[0]
    n_chunks = 4
    PC = P // n_chunks
    hpre = ghb_ref[...] * jnp.ones((1, TN), jnp.float32)
    vT = jnp.zeros((F, TN), jnp.float32)
    for c in range(n_chunks):
        sl = pl.ds(c * PC, PC)
        cic = jnp.tanh(jnp.dot(cmT_ref[sl, :], llT,
                               preferred_element_type=jnp.float32))
        cic_b = cic.astype(jnp.bfloat16)
        hpre = hpre + jnp.dot(ghWT_ref[:, sl], cic_b,
                              preferred_element_type=jnp.float32)
        vT = vT + jnp.dot(outWT_ref[:, sl], cic_b,
                          preferred_element_type=jnp.float32)
    h = jnp.tanh(hpre)
    gate = jnp.sum(h * goW_ref[...], axis=0, keepdims=True) + gob_ref[...]

    giota = jax.lax.broadcasted_iota(jnp.int32, (n_graphs, TN), 0)
    ohG = giota == bi
    ohGf = ohG.astype(jnp.float32)
    tmax = jnp.max(jnp.where(ohG, gate, _NEG), axis=1, keepdims=True)
    m_old = m_s[...]
    m_new = jnp.maximum(m_old, tmax)
    scale = jnp.exp(m_old - m_new)
    mb = jnp.sum(ohGf * m_new, axis=0, keepdims=True)
    e = jnp.exp(gate - mb)
    ohGe = ohGf * e
    d_s[...] = d_s[...] * scale + jnp.sum(ohGe, axis=1, keepdims=True)
    numtile = jax.lax.dot_general(ohGe, vT, (((1,), (1,)), ((), ())),
                                  preferred_element_type=jnp.float32)
    num_s[...] = num_s[...] * scale + numtile
    m_s[...] = m_new

    @pl.when(i == n_tiles - 1)
    def _fin():
        out_ref[...] = num_s[...] / (d_s[...] + 1e-16) + outb_ref[...]


def kernel(x, edge_index, batch, prior, emission, gh_W, gh_b, go_W, go_b,
           out_W, out_b, contrastive):
    del edge_index
    N = x.shape[0]
    G, C = prior.shape
    M = emission.shape[2]
    P = contrastive.shape[1]
    H = gh_W.shape[1]
    F = out_W.shape[1]
    NG = 64

    TN = 2000
    assert N % TN == 0
    NT = N // TN

    x3 = x.astype(jnp.int32).reshape(NT, 1, TN)
    b3 = batch.astype(jnp.int32).reshape(NT, 1, TN)

    f32 = jnp.float32
    body = functools.partial(_cgmn_body, n_tiles=NT, n_graphs=NG)
    out = pl.pallas_call(
        body,
        grid=(NT,),
        in_specs=[
            pl.BlockSpec((1, 1, TN), lambda i: (i, 0, 0)),
            pl.BlockSpec((1, 1, TN), lambda i: (i, 0, 0)),
            pl.BlockSpec((G, C), lambda i: (0, 0)),
            pl.BlockSpec((G, C, M), lambda i: (0, 0, 0)),
            pl.BlockSpec((P, G), lambda i: (0, 0)),
            pl.BlockSpec((H, P), lambda i: (0, 0)),
            pl.BlockSpec((H, 1), lambda i: (0, 0)),
            pl.BlockSpec((H, 1), lambda i: (0, 0)),
            pl.BlockSpec((1, 1), lambda i: (0, 0)),
            pl.BlockSpec((F, P), lambda i: (0, 0)),
            pl.BlockSpec((1, F), lambda i: (0, 0)),
        ],
        out_specs=pl.BlockSpec((NG, F), lambda i: (0, 0)),
        out_shape=jax.ShapeDtypeStruct((NG, F), f32),
        scratch_shapes=[
            pltpu.VMEM((G, M), f32),
            pltpu.VMEM((NG, 1), f32),
            pltpu.VMEM((NG, 1), f32),
            pltpu.VMEM((NG, F), f32),
        ],
    )(x3, b3,
      prior.astype(f32),
      emission.astype(f32),
      contrastive.T.astype(f32),
      gh_W.T.astype(jnp.bfloat16),
      gh_b.reshape(H, 1).astype(f32),
      go_W.reshape(H, 1).astype(f32),
      go_b.reshape(1, 1).astype(f32),
      out_W.T.astype(jnp.bfloat16),
      out_b.reshape(1, F).astype(f32))
    return out


# speedup vs baseline: 1.0206x; 1.0206x over previous
import functools

import jax
import jax.numpy as jnp
from jax.experimental import pallas as pl
from jax.experimental.pallas import tpu as pltpu

_NEG = -1e30


def _cgmn_body(x_ref, b_ref, prior_ref, em_ref, cmT_ref, ghWT_ref,
               ghb_ref, goW_ref, gob_ref, outWT_ref, outb_ref, out_ref,
               T_s, m_s, d_s, num_s, *, n_tiles, n_graphs):
    i = pl.program_id(0)
    G, M = T_s.shape
    F = num_s.shape[1]
    TN = x_ref.shape[2]

    @pl.when(i == 0)
    def _init():
        pr = prior_ref[...]
        pe = jnp.exp(pr - jnp.max(pr, axis=1, keepdims=True))
        p = pe / jnp.sum(pe, axis=1, keepdims=True)
        em = em_ref[...]
        ee = jnp.exp(em - jnp.max(em, axis=2, keepdims=True))
        ep = ee / jnp.sum(ee, axis=2, keepdims=True)
        T_s[...] = jnp.log(jnp.sum(p[:, :, None] * ep, axis=1))
        m_s[...] = jnp.full((n_graphs, 1), _NEG, jnp.float32)
        d_s[...] = jnp.zeros((n_graphs, 1), jnp.float32)
        num_s[...] = jnp.zeros((n_graphs, F), jnp.float32)

    xi = x_ref[0]
    bi = b_ref[0]

    miota = jax.lax.broadcasted_iota(jnp.int32, (M, TN), 0)
    ohM = (miota == xi).astype(jnp.float32)
    llT = jnp.dot(T_s[...], ohM, preferred_element_type=jnp.float32)

    ciT = jnp.tanh(jnp.dot(cmT_ref[...], llT,
                           preferred_element_type=jnp.float32))
    ciT_b = ciT.astype(jnp.bfloat16)
    h = jnp.tanh(jnp.dot(ghWT_ref[...], ciT_b,
                         preferred_element_type=jnp.float32) + ghb_ref[...])
    vT = jnp.dot(outWT_ref[...], ciT_b,
                 preferred_element_type=jnp.float32)
    gate = jnp.sum(h * goW_ref[...], axis=0, keepdims=True) + gob_ref[...]

    giota = jax.lax.broadcasted_iota(jnp.int32, (n_graphs, TN), 0)
    ohG = giota == bi
    ohGf = ohG.astype(jnp.float32)
    tmax = jnp.max(jnp.where(ohG, gate, _NEG), axis=1, keepdims=True)
    m_old = m_s[...]
    m_new = jnp.maximum(m_old, tmax)
    scale = jnp.exp(m_old - m_new)
    mb = jnp.sum(ohGf * m_new, axis=0, keepdims=True)
    e = jnp.exp(gate - mb)
    ohGe = ohGf * e
    d_s[...] = d_s[...] * scale + jnp.sum(ohGe, axis=1, keepdims=True)
    numtile = jax.lax.dot_general(ohGe, vT, (((1,), (1,)), ((), ())),
                                  preferred_element_type=jnp.float32)
    num_s[...] = num_s[...] * scale + numtile
    m_s[...] = m_new

    @pl.when(i == n_tiles - 1)
    def _fin():
        out_ref[...] = num_s[...] / (d_s[...] + 1e-16) + outb_ref[...]


def kernel(x, edge_index, batch, prior, emission, gh_W, gh_b, go_W, go_b,
           out_W, out_b, contrastive):
    del edge_index
    N = x.shape[0]
    G, C = prior.shape
    M = emission.shape[2]
    P = contrastive.shape[1]
    H = gh_W.shape[1]
    F = out_W.shape[1]
    NG = 64

    TN = 2000
    assert N % TN == 0
    NT = N // TN

    x3 = x.astype(jnp.int32).reshape(NT, 1, TN)
    b3 = batch.astype(jnp.int32).reshape(NT, 1, TN)

    f32 = jnp.float32
    body = functools.partial(_cgmn_body, n_tiles=NT, n_graphs=NG)
    out = pl.pallas_call(
        body,
        grid=(NT,),
        in_specs=[
            pl.BlockSpec((1, 1, TN), lambda i: (i, 0, 0)),
            pl.BlockSpec((1, 1, TN), lambda i: (i, 0, 0)),
            pl.BlockSpec((G, C), lambda i: (0, 0)),
            pl.BlockSpec((G, C, M), lambda i: (0, 0, 0)),
            pl.BlockSpec((P, G), lambda i: (0, 0)),
            pl.BlockSpec((H, P), lambda i: (0, 0)),
            pl.BlockSpec((H, 1), lambda i: (0, 0)),
            pl.BlockSpec((H, 1), lambda i: (0, 0)),
            pl.BlockSpec((1, 1), lambda i: (0, 0)),
            pl.BlockSpec((F, P), lambda i: (0, 0)),
            pl.BlockSpec((1, F), lambda i: (0, 0)),
        ],
        out_specs=pl.BlockSpec((NG, F), lambda i: (0, 0)),
        out_shape=jax.ShapeDtypeStruct((NG, F), f32),
        scratch_shapes=[
            pltpu.VMEM((G, M), f32),
            pltpu.VMEM((NG, 1), f32),
            pltpu.VMEM((NG, 1), f32),
            pltpu.VMEM((NG, F), f32),
        ],
    )(x3, b3,
      prior.astype(f32),
      emission.astype(f32),
      contrastive.T.astype(f32),
      gh_W.T.astype(jnp.bfloat16),
      gh_b.reshape(H, 1).astype(f32),
      go_W.reshape(H, 1).astype(f32),
      go_b.reshape(1, 1).astype(f32),
      out_W.T.astype(jnp.bfloat16),
      out_b.reshape(1, F).astype(f32))
    return out


# speedup vs baseline: 1.1425x; 1.1195x over previous
import functools

import jax
import jax.numpy as jnp
from jax.experimental import pallas as pl
from jax.experimental.pallas import tpu as pltpu

_NEG = -1e30


def _cgmn_body(x_ref, b_ref, prior_ref, em_ref, cm_ref, ghW_ref,
               ghb_ref, goW_ref, gob_ref, outW_ref, outb_ref, out_ref,
               T_s, m_s, d_s, num_s, cmT_s, ghWT_s, outWT_s,
               *, n_tiles, n_graphs):
    i = pl.program_id(0)
    G, M = T_s.shape
    F = num_s.shape[1]
    TN = x_ref.shape[2]

    @pl.when(i == 0)
    def _init():
        pr = prior_ref[...]
        pe = jnp.exp(pr - jnp.max(pr, axis=1, keepdims=True))
        p = pe / jnp.sum(pe, axis=1, keepdims=True)
        em = em_ref[...]
        ee = jnp.exp(em - jnp.max(em, axis=2, keepdims=True))
        ep = ee / jnp.sum(ee, axis=2, keepdims=True)
        T_s[...] = jnp.log(jnp.sum(p[:, :, None] * ep, axis=1))
        cmT_s[...] = jnp.transpose(cm_ref[...])
        ghWT_s[...] = jnp.transpose(ghW_ref[...]).astype(jnp.bfloat16)
        outWT_s[...] = jnp.transpose(outW_ref[...]).astype(jnp.bfloat16)
        m_s[...] = jnp.full((n_graphs, 1), _NEG, jnp.float32)
        d_s[...] = jnp.zeros((n_graphs, 1), jnp.float32)
        num_s[...] = jnp.zeros((n_graphs, F), jnp.float32)

    xi = x_ref[0]
    bi = b_ref[0]

    miota = jax.lax.broadcasted_iota(jnp.int32, (M, TN), 0)
    ohM = (miota == xi).astype(jnp.float32)
    llT = jnp.dot(T_s[...], ohM, preferred_element_type=jnp.float32)

    ciT = jnp.tanh(jnp.dot(cmT_s[...], llT,
                           preferred_element_type=jnp.float32))
    ciT_b = ciT.astype(jnp.bfloat16)
    h = jnp.tanh(jnp.dot(ghWT_s[...], ciT_b,
                         preferred_element_type=jnp.float32) + ghb_ref[...])
    vT = jnp.dot(outWT_s[...], ciT_b,
                 preferred_element_type=jnp.float32)
    gate = jnp.sum(h * goW_ref[...], axis=0, keepdims=True) + gob_ref[...]

    giota = jax.lax.broadcasted_iota(jnp.int32, (n_graphs, TN), 0)
    ohG = giota == bi
    ohGf = ohG.astype(jnp.float32)
    tmax = jnp.max(jnp.where(ohG, gate, _NEG), axis=1, keepdims=True)
    m_old = m_s[...]
    m_new = jnp.maximum(m_old, tmax)
    scale = jnp.exp(m_old - m_new)
    mb = jnp.sum(ohGf * m_new, axis=0, keepdims=True)
    e = jnp.exp(gate - mb)
    ohGe = ohGf * e
    d_s[...] = d_s[...] * scale + jnp.sum(ohGe, axis=1, keepdims=True)
    numtile = jax.lax.dot_general(ohGe, vT, (((1,), (1,)), ((), ())),
                                  preferred_element_type=jnp.float32)
    num_s[...] = num_s[...] * scale + numtile
    m_s[...] = m_new

    @pl.when(i == n_tiles - 1)
    def _fin():
        out_ref[...] = num_s[...] / (d_s[...] + 1e-16) + outb_ref[...]


def kernel(x, edge_index, batch, prior, emission, gh_W, gh_b, go_W, go_b,
           out_W, out_b, contrastive):
    del edge_index
    N = x.shape[0]
    G, C = prior.shape
    M = emission.shape[2]
    P = contrastive.shape[1]
    H = gh_W.shape[1]
    F = out_W.shape[1]
    NG = 64

    TN = 2000
    assert N % TN == 0
    NT = N // TN

    x3 = x.astype(jnp.int32).reshape(NT, 1, TN)
    b3 = batch.astype(jnp.int32).reshape(NT, 1, TN)

    f32 = jnp.float32
    body = functools.partial(_cgmn_body, n_tiles=NT, n_graphs=NG)
    out = pl.pallas_call(
        body,
        grid=(NT,),
        in_specs=[
            pl.BlockSpec((1, 1, TN), lambda i: (i, 0, 0)),
            pl.BlockSpec((1, 1, TN), lambda i: (i, 0, 0)),
            pl.BlockSpec((G, C), lambda i: (0, 0)),
            pl.BlockSpec((G, C, M), lambda i: (0, 0, 0)),
            pl.BlockSpec((G, P), lambda i: (0, 0)),
            pl.BlockSpec((P, H), lambda i: (0, 0)),
            pl.BlockSpec((H, 1), lambda i: (0, 0)),
            pl.BlockSpec((H, 1), lambda i: (0, 0)),
            pl.BlockSpec((1, 1), lambda i: (0, 0)),
            pl.BlockSpec((P, F), lambda i: (0, 0)),
            pl.BlockSpec((1, F), lambda i: (0, 0)),
        ],
        out_specs=pl.BlockSpec((NG, F), lambda i: (0, 0)),
        out_shape=jax.ShapeDtypeStruct((NG, F), f32),
        scratch_shapes=[
            pltpu.VMEM((G, M), f32),
            pltpu.VMEM((NG, 1), f32),
            pltpu.VMEM((NG, 1), f32),
            pltpu.VMEM((NG, F), f32),
            pltpu.VMEM((P, G), f32),
            pltpu.VMEM((H, P), jnp.bfloat16),
            pltpu.VMEM((F, P), jnp.bfloat16),
        ],
    )(x3, b3,
      prior.astype(f32),
      emission.astype(f32),
      contrastive.astype(f32),
      gh_W.astype(f32),
      gh_b.reshape(H, 1).astype(f32),
      go_W.reshape(H, 1).astype(f32),
      go_b.reshape(1, 1).astype(f32),
      out_W.astype(f32),
      out_b.reshape(1, F).astype(f32))
    return out


# speedup vs baseline: 1.1800x; 1.0328x over previous
import functools

import jax
import jax.numpy as jnp
from jax.experimental import pallas as pl
from jax.experimental.pallas import tpu as pltpu

_NEG = -1e30


def _cgmn_body(x_ref, b_ref, prior_ref, em_ref, cm_ref, ghW_ref,
               ghb_ref, goW_ref, gob_ref, outW_ref, outb_ref, out_ref,
               W_s, m_s, d_s, num_s, ghWT_s, outWT_s,
               *, n_tiles, n_graphs):
    i = pl.program_id(0)
    M = W_s.shape[1]
    F = num_s.shape[1]
    TN = x_ref.shape[2]

    @pl.when(i == 0)
    def _init():
        pr = prior_ref[...]
        pe = jnp.exp(pr - jnp.max(pr, axis=1, keepdims=True))
        p = pe / jnp.sum(pe, axis=1, keepdims=True)
        em = em_ref[...]
        ee = jnp.exp(em - jnp.max(em, axis=2, keepdims=True))
        ep = ee / jnp.sum(ee, axis=2, keepdims=True)
        T = jnp.log(jnp.sum(p[:, :, None] * ep, axis=1))
        cmT = jnp.transpose(cm_ref[...])
        W_s[...] = jnp.dot(cmT, T,
                           preferred_element_type=jnp.float32).astype(jnp.bfloat16)
        ghWT_s[...] = jnp.transpose(ghW_ref[...]).astype(jnp.bfloat16)
        outWT_s[...] = jnp.transpose(outW_ref[...]).astype(jnp.bfloat16)
        m_s[...] = jnp.full((n_graphs, 1), _NEG, jnp.float32)
        d_s[...] = jnp.zeros((n_graphs, 1), jnp.float32)
        num_s[...] = jnp.zeros((n_graphs, F), jnp.float32)

    xi = x_ref[0]
    bi = b_ref[0]

    miota = jax.lax.broadcasted_iota(jnp.int32, (M, TN), 0)
    ohM = (miota == xi).astype(jnp.bfloat16)
    ciT = jnp.tanh(jnp.dot(W_s[...], ohM,
                           preferred_element_type=jnp.float32))
    ciT_b = ciT.astype(jnp.bfloat16)
    h = jnp.tanh(jnp.dot(ghWT_s[...], ciT_b,
                         preferred_element_type=jnp.float32) + ghb_ref[...])
    vT = jnp.dot(outWT_s[...], ciT_b,
                 preferred_element_type=jnp.float32)
    gate = jnp.sum(h * goW_ref[...], axis=0, keepdims=True) + gob_ref[...]

    giota = jax.lax.broadcasted_iota(jnp.int32, (n_graphs, TN), 0)
    ohG = giota == bi
    ohGf = ohG.astype(jnp.float32)
    tmax = jnp.max(jnp.where(ohG, gate, _NEG), axis=1, keepdims=True)
    m_old = m_s[...]
    m_new = jnp.maximum(m_old, tmax)
    scale = jnp.exp(m_old - m_new)
    mb = jnp.sum(ohGf * m_new, axis=0, keepdims=True)
    e = jnp.exp(gate - mb)
    ohGe = ohGf * e
    d_s[...] = d_s[...] * scale + jnp.sum(ohGe, axis=1, keepdims=True)
    numtile = jax.lax.dot_general(ohGe, vT, (((1,), (1,)), ((), ())),
                                  preferred_element_type=jnp.float32)
    num_s[...] = num_s[...] * scale + numtile
    m_s[...] = m_new

    @pl.when(i == n_tiles - 1)
    def _fin():
        out_ref[...] = num_s[...] / (d_s[...] + 1e-16) + outb_ref[...]


def kernel(x, edge_index, batch, prior, emission, gh_W, gh_b, go_W, go_b,
           out_W, out_b, contrastive):
    del edge_index
    N = x.shape[0]
    G, C = prior.shape
    M = emission.shape[2]
    P = contrastive.shape[1]
    H = gh_W.shape[1]
    F = out_W.shape[1]
    NG = 64

    TN = 2000
    assert N % TN == 0
    NT = N // TN

    x3 = x.astype(jnp.int32).reshape(NT, 1, TN)
    b3 = batch.astype(jnp.int32).reshape(NT, 1, TN)

    f32 = jnp.float32
    body = functools.partial(_cgmn_body, n_tiles=NT, n_graphs=NG)
    out = pl.pallas_call(
        body,
        grid=(NT,),
        in_specs=[
            pl.BlockSpec((1, 1, TN), lambda i: (i, 0, 0)),
            pl.BlockSpec((1, 1, TN), lambda i: (i, 0, 0)),
            pl.BlockSpec((G, C), lambda i: (0, 0)),
            pl.BlockSpec((G, C, M), lambda i: (0, 0, 0)),
            pl.BlockSpec((G, P), lambda i: (0, 0)),
            pl.BlockSpec((P, H), lambda i: (0, 0)),
            pl.BlockSpec((H, 1), lambda i: (0, 0)),
            pl.BlockSpec((H, 1), lambda i: (0, 0)),
            pl.BlockSpec((1, 1), lambda i: (0, 0)),
            pl.BlockSpec((P, F), lambda i: (0, 0)),
            pl.BlockSpec((1, F), lambda i: (0, 0)),
        ],
        out_specs=pl.BlockSpec((NG, F), lambda i: (0, 0)),
        out_shape=jax.ShapeDtypeStruct((NG, F), f32),
        scratch_shapes=[
            pltpu.VMEM((P, M), jnp.bfloat16),
            pltpu.VMEM((NG, 1), f32),
            pltpu.VMEM((NG, 1), f32),
            pltpu.VMEM((NG, F), f32),
            pltpu.VMEM((H, P), jnp.bfloat16),
            pltpu.VMEM((F, P), jnp.bfloat16),
        ],
    )(x3, b3,
      prior.astype(f32),
      emission.astype(f32),
      contrastive.astype(f32),
      gh_W.astype(f32),
      gh_b.reshape(H, 1).astype(f32),
      go_W.reshape(H, 1).astype(f32),
      go_b.reshape(1, 1).astype(f32),
      out_W.astype(f32),
      out_b.reshape(1, F).astype(f32))
    return out


# speedup vs baseline: 3.4567x; 2.9293x over previous
import functools

import jax
import jax.numpy as jnp
from jax.experimental import pallas as pl
from jax.experimental.pallas import tpu as pltpu

_NEG = -1e30


def _cgmn_body(x_ref, b_ref, prior_ref, em_ref, cm_ref, ghW_ref,
               ghb_ref, goW_ref, gob_ref, outW_ref, outb_ref, out_ref,
               *, n_graphs):
    M = em_ref.shape[2]
    TN = x_ref.shape[2]

    pr = prior_ref[...]
    pe = jnp.exp(pr - jnp.max(pr, axis=1, keepdims=True))
    p = pe / jnp.sum(pe, axis=1, keepdims=True)
    em = em_ref[...]
    ee = jnp.exp(em - jnp.max(em, axis=2, keepdims=True))
    ep = ee / jnp.sum(ee, axis=2, keepdims=True)
    T = jnp.log(jnp.sum(p[:, :, None] * ep, axis=1))
    cmT = jnp.transpose(cm_ref[...])
    W = jnp.dot(cmT, T, preferred_element_type=jnp.float32)
    tw = jnp.tanh(W)
    tw_b = tw.astype(jnp.bfloat16)
    ghWT = jnp.transpose(ghW_ref[...]).astype(jnp.bfloat16)
    outWT = jnp.transpose(outW_ref[...]).astype(jnp.bfloat16)
    htab = jnp.tanh(jnp.dot(ghWT, tw_b,
                            preferred_element_type=jnp.float32) + ghb_ref[...])
    gate_tab = (jnp.sum(htab * goW_ref[...], axis=0, keepdims=True)
                + gob_ref[...])
    vtab = jnp.dot(outWT, tw_b,
                   preferred_element_type=jnp.float32)

    xi = x_ref[0]
    bi = b_ref[0]
    miota = jax.lax.broadcasted_iota(jnp.int32, (M, TN), 0)
    ohM = (miota == xi).astype(jnp.bfloat16)
    giota = jax.lax.broadcasted_iota(jnp.int32, (n_graphs, TN), 0)
    ohG = (giota == bi).astype(jnp.bfloat16)
    cnt = jax.lax.dot_general(ohG, ohM, (((1,), (1,)), ((), ())),
                              preferred_element_type=jnp.float32)

    present = cnt > 0.0
    gmax = jnp.max(jnp.where(present, gate_tab, _NEG), axis=1,
                   keepdims=True)
    etab = jnp.where(present, jnp.exp(gate_tab - gmax), 0.0) * cnt
    denom = jnp.sum(etab, axis=1, keepdims=True)
    num = jax.lax.dot_general(etab, vtab, (((1,), (1,)), ((), ())),
                              preferred_element_type=jnp.float32)
    out_ref[...] = num / (denom + 1e-16) + outb_ref[...]


def kernel(x, edge_index, batch, prior, emission, gh_W, gh_b, go_W, go_b,
           out_W, out_b, contrastive):
    del edge_index
    N = x.shape[0]
    G, C = prior.shape
    M = emission.shape[2]
    P = contrastive.shape[1]
    H = gh_W.shape[1]
    F = out_W.shape[1]
    NG = 64

    TN = -(-N // 512) * 512
    x32 = x.astype(jnp.int32)
    b32 = batch.astype(jnp.int32)
    xp = jnp.concatenate([x32, jnp.zeros((TN - N,), jnp.int32)])
    bp = jnp.concatenate([b32, jnp.full((TN - N,), NG, jnp.int32)])
    x3 = xp.reshape(1, 1, TN)
    b3 = bp.reshape(1, 1, TN)

    f32 = jnp.float32
    body = functools.partial(_cgmn_body, n_graphs=NG)
    out = pl.pallas_call(
        body,
        grid=(1,),
        in_specs=[
            pl.BlockSpec((1, 1, TN), lambda i: (0, 0, 0)),
            pl.BlockSpec((1, 1, TN), lambda i: (0, 0, 0)),
            pl.BlockSpec((G, C), lambda i: (0, 0)),
            pl.BlockSpec((G, C, M), lambda i: (0, 0, 0)),
            pl.BlockSpec((G, P), lambda i: (0, 0)),
            pl.BlockSpec((P, H), lambda i: (0, 0)),
            pl.BlockSpec((H, 1), lambda i: (0, 0)),
            pl.BlockSpec((H, 1), lambda i: (0, 0)),
            pl.BlockSpec((1, 1), lambda i: (0, 0)),
            pl.BlockSpec((P, F), lambda i: (0, 0)),
            pl.BlockSpec((1, F), lambda i: (0, 0)),
        ],
        out_specs=pl.BlockSpec((NG, F), lambda i: (0, 0)),
        out_shape=jax.ShapeDtypeStruct((NG, F), f32),
    )(x3, b3,
      prior.astype(f32),
      emission.astype(f32),
      contrastive.astype(f32),
      gh_W.astype(f32),
      gh_b.reshape(H, 1).astype(f32),
      go_W.reshape(H, 1).astype(f32),
      go_b.reshape(1, 1).astype(f32),
      out_W.astype(f32),
      out_b.reshape(1, F).astype(f32))
    return out


# speedup vs baseline: 3.6702x; 1.0618x over previous
import functools

import jax
import jax.numpy as jnp
from jax.experimental import pallas as pl
from jax.experimental.pallas import tpu as pltpu

_NEG = -1e30


def _cgmn_body(x_ref, b_ref, prior_ref, em_ref, cm_ref, ghW_ref,
               ghb_ref, goW_ref, gob_ref, outW_ref, outb_ref, out_ref,
               *, n_graphs):
    M = em_ref.shape[2]
    TN = x_ref.shape[2]

    pr = prior_ref[...]
    pe = jnp.exp(pr - jnp.max(pr, axis=1, keepdims=True))
    p = pe / jnp.sum(pe, axis=1, keepdims=True)
    em = em_ref[...]
    ee = jnp.exp(em - jnp.max(em, axis=2, keepdims=True))
    ep = ee / jnp.sum(ee, axis=2, keepdims=True)
    T = jnp.log(jnp.sum(p[:, :, None] * ep, axis=1))
    cmT = jnp.transpose(cm_ref[...])
    W = jnp.dot(cmT, T, preferred_element_type=jnp.float32)
    tw = jnp.tanh(W)
    tw_b = tw.astype(jnp.bfloat16)
    ghWT = jnp.transpose(ghW_ref[...]).astype(jnp.bfloat16)
    outWT = jnp.transpose(outW_ref[...]).astype(jnp.bfloat16)
    htab = jnp.tanh(jnp.dot(ghWT, tw_b,
                            preferred_element_type=jnp.float32) + ghb_ref[...])
    gate_tab = (jnp.sum(htab * goW_ref[...], axis=0, keepdims=True)
                + gob_ref[...])
    vtab = jnp.dot(outWT, tw_b,
                   preferred_element_type=jnp.float32)

    xi = x_ref[0]
    bi = b_ref[0]
    miota = jax.lax.broadcasted_iota(jnp.int32, (M, TN), 0)
    ohM = (miota == xi).astype(jnp.bfloat16)
    giota = jax.lax.broadcasted_iota(jnp.int32, (n_graphs, TN), 0)
    ohG = (giota == bi).astype(jnp.bfloat16)
    cnt = jax.lax.dot_general(ohG, ohM, (((1,), (1,)), ((), ())),
                              preferred_element_type=jnp.float32)

    present = cnt > 0.0
    gmax = jnp.max(jnp.where(present, gate_tab, _NEG), axis=1,
                   keepdims=True)
    etab = jnp.where(present, jnp.exp(gate_tab - gmax), 0.0) * cnt
    denom = jnp.sum(etab, axis=1, keepdims=True)
    num = jax.lax.dot_general(etab, vtab, (((1,), (1,)), ((), ())),
                              preferred_element_type=jnp.float32)
    out_ref[...] = num / (denom + 1e-16) + outb_ref[...]


def kernel(x, edge_index, batch, prior, emission, gh_W, gh_b, go_W, go_b,
           out_W, out_b, contrastive):
    del edge_index
    N = x.shape[0]
    G, C = prior.shape
    M = emission.shape[2]
    P = contrastive.shape[1]
    H = gh_W.shape[1]
    F = out_W.shape[1]
    NG = 64

    TN = N
    x3 = x.astype(jnp.int32).reshape(1, 1, TN)
    b3 = batch.astype(jnp.int32).reshape(1, 1, TN)

    f32 = jnp.float32
    body = functools.partial(_cgmn_body, n_graphs=NG)
    out = pl.pallas_call(
        body,
        grid=(1,),
        in_specs=[
            pl.BlockSpec((1, 1, TN), lambda i: (0, 0, 0)),
            pl.BlockSpec((1, 1, TN), lambda i: (0, 0, 0)),
            pl.BlockSpec((G, C), lambda i: (0, 0)),
            pl.BlockSpec((G, C, M), lambda i: (0, 0, 0)),
            pl.BlockSpec((G, P), lambda i: (0, 0)),
            pl.BlockSpec((P, H), lambda i: (0, 0)),
            pl.BlockSpec((H, 1), lambda i: (0, 0)),
            pl.BlockSpec((H, 1), lambda i: (0, 0)),
            pl.BlockSpec((1, 1), lambda i: (0, 0)),
            pl.BlockSpec((P, F), lambda i: (0, 0)),
            pl.BlockSpec((1, F), lambda i: (0, 0)),
        ],
        out_specs=pl.BlockSpec((NG, F), lambda i: (0, 0)),
        out_shape=jax.ShapeDtypeStruct((NG, F), f32),
    )(x3, b3,
      prior.astype(f32),
      emission.astype(f32),
      contrastive.astype(f32),
      gh_W.astype(f32),
      gh_b.reshape(H, 1).astype(f32),
      go_W.reshape(H, 1).astype(f32),
      go_b.reshape(1, 1).astype(f32),
      out_W.astype(f32),
      out_b.reshape(1, F).astype(f32))
    return out
